# baseline (speedup 1.0000x reference)
import functools

import jax
import jax.numpy as jnp
from jax import lax
from jax.experimental import pallas as pl
from jax.experimental.pallas import tpu as pltpu
from jax.experimental.pallas import tpu_sc as plsc

_COST = 0.25
_MT = 256


_CW = 2048


def _argmin_body(z_ref, z2_ref, w_ref, d2_ref, idx_ref, loss_ref):
    mt, k = z_ref.shape[0], w_ref.shape[0]
    z = z_ref[...]
    z2 = z2_ref[...]
    dn = (((1,), (1,)), ((), ()))
    rmin = None
    ridx = None
    for j in range(k // _CW):
        wj = w_ref[j * _CW : (j + 1) * _CW, :]
        e = lax.dot_general(z, wj, dn, preferred_element_type=jnp.float32)
        dist = (z2 + d2_ref[:, j * _CW : (j + 1) * _CW]) - 2.0 * e
        if j == 0:
            rmin = dist
            ridx = jnp.zeros((mt, _CW), jnp.int32)
        else:
            lt = dist < rmin
            rmin = jnp.where(lt, dist, rmin)
            ridx = jnp.where(lt, j, ridx)
    tmin = jnp.min(rmin, axis=1, keepdims=True)
    lane = lax.broadcasted_iota(jnp.int32, (mt, _CW), 1)
    cand = jnp.where(rmin == tmin, ridx * _CW + lane, k)
    idx_ref[...] = jnp.min(cand, axis=1, keepdims=True)
    part = jnp.sum(tmin, axis=(0, 1), keepdims=True)
    i = pl.program_id(0)

    @pl.when(i == 0)
    def _():
        loss_ref[...] = part

    @pl.when(i > 0)
    def _():
        loss_ref[...] += part


def _argmin_call(zf, z2c, wk, d2r, interpret=False):
    t, c = zf.shape
    k = wk.shape[0]
    return pl.pallas_call(
        _argmin_body,
        grid=(t // _MT,),
        in_specs=[
            pl.BlockSpec((_MT, c), lambda i: (i, 0)),
            pl.BlockSpec((_MT, 1), lambda i: (i, 0)),
            pl.BlockSpec((k, c), lambda i: (0, 0)),
            pl.BlockSpec((1, k), lambda i: (0, 0)),
        ],
        out_specs=[
            pl.BlockSpec((_MT, 1), lambda i: (i, 0)),
            pl.BlockSpec((1, 1), lambda i: (0, 0)),
        ],
        out_shape=[
            jax.ShapeDtypeStruct((t, 1), jnp.int32),
            jax.ShapeDtypeStruct((1, 1), jnp.float32),
        ],
        interpret=interpret,
    )(zf, z2c, wk, d2r)


@functools.cache
def _make_gather(t, c):
    info = plsc.get_sparse_core_info()
    nw = info.num_cores * info.num_subcores
    bpw = t // nw
    mesh = plsc.VectorSubcoreMesh(core_axis_name="c", subcore_axis_name="s")

    @functools.partial(
        pl.kernel,
        mesh=mesh,
        compiler_params=pltpu.CompilerParams(use_tc_tiling_on_sc=False),
        out_type=jax.ShapeDtypeStruct((t, c), jnp.float32),
        scratch_types=[
            pltpu.VMEM((bpw,), jnp.int32),
            pltpu.VMEM((bpw, c), jnp.float32),
            pltpu.SemaphoreType.DMA,
        ],
    )
    def gather_k(table_hbm, idx_hbm, out_hbm, idx_v, rows_v, sem):
        wid = lax.axis_index("s") * info.num_cores + lax.axis_index("c")
        base = wid * bpw
        pltpu.sync_copy(idx_hbm.at[pl.ds(base, bpw)], idx_v)
        pltpu.async_copy(table_hbm.at[idx_v], rows_v, sem).wait()
        pltpu.sync_copy(rows_v, out_hbm.at[pl.ds(base, bpw)])

    return gather_k


def kernel(z, W):
    b, c, h, w = z.shape
    k = W.shape[0]
    hw = h * w
    t = b * hw
    z_flat = jnp.transpose(z.reshape(b, c, hw), (0, 2, 1))
    z2 = jnp.sum(z_flat * z_flat, axis=-1)
    d2 = jnp.sum(W * W, axis=-1)

    idx2, loss_sum = _argmin_call(
        z_flat.reshape(t, c), z2.reshape(t, 1), W, d2.reshape(1, k)
    )
    indices = idx2.reshape(t)
    quant_flat = _make_gather(t, c)(W, indices)
    quantized = jnp.transpose(quant_flat.reshape(b, hw, c), (0, 2, 1)).reshape(
        b, c, h, w
    )
    loss = loss_sum[0, 0] * jnp.float32(_COST / (t * c))
    return indices.reshape(b, h, w), quantized, loss


# speedup vs baseline: 1.0875x; 1.0875x over previous
import functools

import jax
import jax.numpy as jnp
from jax import lax
from jax.experimental import pallas as pl
from jax.experimental.pallas import tpu as pltpu
from jax.experimental.pallas import tpu_sc as plsc

_COST = 0.25
_MT = 256


_CW = 2048


def _argmin_body(z_ref, z2_ref, wt_ref, d2_ref, idx_ref, loss_ref):
    mt, k = z_ref.shape[0], wt_ref.shape[1]
    z = z_ref[...]
    z2 = z2_ref[...]
    rmin = None
    ridx = None
    for j in range(k // _CW):
        wj = wt_ref[:, j * _CW : (j + 1) * _CW]
        e = jnp.dot(z, wj, preferred_element_type=jnp.float32)
        dist = (z2 + d2_ref[:, j * _CW : (j + 1) * _CW]) - 2.0 * e
        if j == 0:
            rmin = dist
            ridx = jnp.zeros((mt, _CW), jnp.float32)
        else:
            lt = dist < rmin
            rmin = jnp.minimum(dist, rmin)
            ridx = jnp.where(lt, jnp.float32(j), ridx)
    tmin = jnp.min(rmin, axis=1, keepdims=True)
    lane = lax.broadcasted_iota(jnp.int32, (mt, _CW), 1).astype(jnp.float32)
    cand = jnp.where(rmin == tmin, ridx * jnp.float32(_CW) + lane, jnp.float32(k))
    idx_ref[...] = jnp.min(cand, axis=1, keepdims=True).astype(jnp.int32)
    part = jnp.sum(tmin, axis=(0, 1), keepdims=True)
    i = pl.program_id(0)

    @pl.when(i == 0)
    def _():
        loss_ref[...] = part

    @pl.when(i > 0)
    def _():
        loss_ref[...] += part


def _argmin_call(zf, z2c, wt, d2r, interpret=False):
    t, c = zf.shape
    k = wt.shape[1]
    return pl.pallas_call(
        _argmin_body,
        grid=(t // _MT,),
        in_specs=[
            pl.BlockSpec((_MT, c), lambda i: (i, 0)),
            pl.BlockSpec((_MT, 1), lambda i: (i, 0)),
            pl.BlockSpec((c, k), lambda i: (0, 0)),
            pl.BlockSpec((1, k), lambda i: (0, 0)),
        ],
        out_specs=[
            pl.BlockSpec((_MT, 1), lambda i: (i, 0)),
            pl.BlockSpec((1, 1), lambda i: (0, 0)),
        ],
        out_shape=[
            jax.ShapeDtypeStruct((t, 1), jnp.int32),
            jax.ShapeDtypeStruct((1, 1), jnp.float32),
        ],
        interpret=interpret,
    )(zf, z2c, wt, d2r)


@functools.cache
def _make_gather(t, c):
    info = plsc.get_sparse_core_info()
    nw = info.num_cores * info.num_subcores
    bpw = t // nw
    mesh = plsc.VectorSubcoreMesh(core_axis_name="c", subcore_axis_name="s")

    @functools.partial(
        pl.kernel,
        mesh=mesh,
        compiler_params=pltpu.CompilerParams(use_tc_tiling_on_sc=False),
        out_type=jax.ShapeDtypeStruct((t, c), jnp.float32),
        scratch_types=[
            pltpu.VMEM((bpw,), jnp.int32),
            pltpu.VMEM((bpw, c), jnp.float32),
            pltpu.SemaphoreType.DMA,
        ],
    )
    def gather_k(table_hbm, idx_hbm, out_hbm, idx_v, rows_v, sem):
        wid = lax.axis_index("s") * info.num_cores + lax.axis_index("c")
        base = wid * bpw
        pltpu.sync_copy(idx_hbm.at[pl.ds(base, bpw)], idx_v)
        pltpu.async_copy(table_hbm.at[idx_v], rows_v, sem).wait()
        pltpu.sync_copy(rows_v, out_hbm.at[pl.ds(base, bpw)])

    return gather_k


def kernel(z, W):
    b, c, h, w = z.shape
    k = W.shape[0]
    hw = h * w
    t = b * hw
    z_flat = jnp.transpose(z.reshape(b, c, hw), (0, 2, 1))
    z2 = jnp.sum(z_flat * z_flat, axis=-1)
    d2 = jnp.sum(W * W, axis=-1)

    idx2, loss_sum = _argmin_call(
        z_flat.reshape(t, c), z2.reshape(t, 1), W.T, d2.reshape(1, k)
    )
    indices = idx2.reshape(t)
    quant_flat = _make_gather(t, c)(W, indices)
    quantized = jnp.transpose(quant_flat.reshape(b, hw, c), (0, 2, 1)).reshape(
        b, c, h, w
    )
    loss = loss_sum[0, 0] * jnp.float32(_COST / (t * c))
    return indices.reshape(b, h, w), quantized, loss
